# Initial kernel scaffold; baseline (speedup 1.0000x reference)
import jax
import jax.numpy as jnp
from jax.experimental import pallas as pl


def kernel(hidden_states, ln1_gamma, ln2_gamma, Wq, Wk, Wv, Wo, q_norm_gamma, k_norm_gamma, gate_w):
    raise NotImplementedError("write your pallas kernel here")


# speedup vs baseline: 1.1827x; 1.1827x over previous
import jax
import jax.numpy as jnp
import numpy as np
from jax import lax
from jax.experimental import pallas as pl

D_MODEL = 2048
N_HEADS = 16
N_KV_HEADS = 4
HEAD_DIM = 128
N_EXPERTS = 64
TOP_K = 8
EPS = 1e-6
ROPE_THETA = 1000000.0
T = 2048
BT = 256
HALF = HEAD_DIM // 2
BF16 = jnp.bfloat16


def _qkv_kernel(x_ref, w_ref, cos_ref, sin_ref, qg_ref, kg_ref,
                o_ref):
    j = pl.program_id(0)
    xn = x_ref[...]
    y = lax.dot_general(w_ref[...].astype(BF16), xn.astype(BF16),
                        (((0,), (0,)), ((), ())),
                        preferred_element_type=jnp.float32)
    cosv = cos_ref[...]
    sinv = sin_ref[...]

    def headnorm_rope(c, g):
        ms = jnp.mean(c * c, axis=0, keepdims=True)
        cn = c * lax.rsqrt(ms + EPS) * g
        x1 = cn[:HALF, :]
        x2 = cn[HALF:, :]
        return jnp.concatenate(
            [x1 * cosv - x2 * sinv, x2 * cosv + x1 * sinv], axis=0)

    @pl.when(j < 2)
    def _():
        for h in range(8):
            sl = slice(h * HEAD_DIM, (h + 1) * HEAD_DIM)
            o_ref[sl, :] = headnorm_rope(y[sl, :], qg_ref[...])

    @pl.when(j == 2)
    def _():
        for h in range(4):
            sl = slice(h * HEAD_DIM, (h + 1) * HEAD_DIM)
            o_ref[sl, :] = headnorm_rope(y[sl, :], kg_ref[...])
        o_ref[512:, :] = y[512:, :]


def _attn_kernel(q_ref, k_ref, v_ref, o_ref):
    qi = pl.program_id(1)
    st = lax.dot_general(k_ref[...].astype(BF16), q_ref[...].astype(BF16),
                         (((0,), (0,)), ((), ())),
                         preferred_element_type=jnp.float32)
    st = st / np.sqrt(HEAD_DIM)
    srow = lax.broadcasted_iota(jnp.int32, (T, BT), 0)
    tcol = qi * BT + lax.broadcasted_iota(jnp.int32, (T, BT), 1)
    st = jnp.where(srow <= tcol, st, -1e30)
    m = jnp.max(st, axis=0, keepdims=True)
    p = jnp.exp(st - m)
    l = jnp.sum(p, axis=0, keepdims=True)
    attn = p * (1.0 / l)
    o_ref[...] = lax.dot_general(v_ref[...].astype(BF16), attn.astype(BF16),
                                 (((1,), (0,)), ((), ())),
                                 preferred_element_type=jnp.float32)


def _proj_router_kernel(ctx_ref, res_ref, wo_ref, g2_ref, gw_ref,
                        r_ref, w_ref, id_ref):
    attn_out = lax.dot_general(wo_ref[...].astype(BF16),
                               ctx_ref[...].astype(BF16),
                               (((0,), (0,)), ((), ())),
                               preferred_element_type=jnp.float32)
    res2 = res_ref[...] + attn_out
    r_ref[...] = res2
    var = jnp.mean(res2 * res2, axis=0, keepdims=True)
    h2 = res2 * lax.rsqrt(var + EPS) * g2_ref[...]
    logits = lax.dot_general(gw_ref[...].astype(BF16), h2.astype(BF16),
                             (((0,), (0,)), ((), ())),
                             preferred_element_type=jnp.float32)
    iota = lax.broadcasted_iota(jnp.int32, (N_EXPERTS, BT), 0)
    cur = logits
    ws, ids = [], []
    for _ in range(TOP_K):
        m = jnp.max(cur, axis=0, keepdims=True)
        idx = jnp.min(jnp.where(cur == m, iota, N_EXPERTS), axis=0,
                      keepdims=True)
        ws.append(m)
        ids.append(idx)
        cur = jnp.where(iota == idx, -1e30, cur)
    wcat = jnp.concatenate(ws, axis=0)
    e = jnp.exp(wcat - wcat[0:1, :])
    w_ref[...] = e / jnp.sum(e, axis=0, keepdims=True)
    id_ref[...] = jnp.concatenate(ids, axis=0)


def kernel(hidden_states, ln1_gamma, ln2_gamma, Wq, Wk, Wv, Wo,
           q_norm_gamma, k_norm_gamma, gate_w):
    f32 = jnp.float32
    xt = hidden_states.T
    var1 = jnp.mean(jnp.square(hidden_states), axis=-1, keepdims=True)
    xnt = (hidden_states * lax.rsqrt(var1 + EPS) * ln1_gamma).T
    wqkv = jnp.concatenate([Wq, Wk, Wv], axis=1)
    g2 = ln2_gamma.reshape(D_MODEL, 1)
    qg = q_norm_gamma.reshape(HEAD_DIM, 1)
    kg = k_norm_gamma.reshape(HEAD_DIM, 1)
    inv_freq = 1.0 / (ROPE_THETA ** (jnp.arange(0, HALF, dtype=f32) / HALF))
    pos = jnp.arange(T, dtype=f32)
    ang = pos[:, None] * inv_freq[None, :]
    cos_t = jnp.cos(ang).T
    sin_t = jnp.sin(ang).T

    yt = pl.pallas_call(
        _qkv_kernel,
        grid=(3, T // BT),
        in_specs=[
            pl.BlockSpec((D_MODEL, BT), lambda j, i: (0, i)),
            pl.BlockSpec((D_MODEL, 1024), lambda j, i: (0, j)),
            pl.BlockSpec((HALF, BT), lambda j, i: (0, i)),
            pl.BlockSpec((HALF, BT), lambda j, i: (0, i)),
            pl.BlockSpec((HEAD_DIM, 1), lambda j, i: (0, 0)),
            pl.BlockSpec((HEAD_DIM, 1), lambda j, i: (0, 0)),
        ],
        out_specs=pl.BlockSpec((1024, BT), lambda j, i: (j, i)),
        out_shape=jax.ShapeDtypeStruct((3072, T), f32),
    )(xnt, wqkv, cos_t, sin_t, qg, kg)

    ctx_t = pl.pallas_call(
        _attn_kernel,
        grid=(N_HEADS, T // BT),
        in_specs=[
            pl.BlockSpec((HEAD_DIM, BT), lambda h, i: (h, i)),
            pl.BlockSpec((HEAD_DIM, T), lambda h, i: (16 + h // 4, 0)),
            pl.BlockSpec((HEAD_DIM, T), lambda h, i: (20 + h // 4, 0)),
        ],
        out_specs=pl.BlockSpec((HEAD_DIM, BT), lambda h, i: (h, i)),
        out_shape=jax.ShapeDtypeStruct((N_HEADS * HEAD_DIM, T), f32),
    )(yt, yt, yt)

    res2_t, topk_w, topk_ids = pl.pallas_call(
        _proj_router_kernel,
        grid=(T // BT,),
        in_specs=[
            pl.BlockSpec((N_HEADS * HEAD_DIM, BT), lambda i: (0, i)),
            pl.BlockSpec((D_MODEL, BT), lambda i: (0, i)),
            pl.BlockSpec((N_HEADS * HEAD_DIM, D_MODEL), lambda i: (0, 0)),
            pl.BlockSpec((D_MODEL, 1), lambda i: (0, 0)),
            pl.BlockSpec((D_MODEL, N_EXPERTS), lambda i: (0, 0)),
        ],
        out_specs=[
            pl.BlockSpec((D_MODEL, BT), lambda i: (0, i)),
            pl.BlockSpec((TOP_K, BT), lambda i: (0, i)),
            pl.BlockSpec((TOP_K, BT), lambda i: (0, i)),
        ],
        out_shape=[
            jax.ShapeDtypeStruct((D_MODEL, T), f32),
            jax.ShapeDtypeStruct((TOP_K, T), f32),
            jax.ShapeDtypeStruct((TOP_K, T), jnp.int32),
        ],
    )(ctx_t, xt, Wo, g2, gate_w)

    return (res2_t.T, topk_w.T, topk_ids.T.astype(jnp.int64))
